# Initial kernel scaffold; baseline (speedup 1.0000x reference)
import jax
import jax.numpy as jnp
from jax.experimental import pallas as pl


def kernel(pred, gt, partial):
    raise NotImplementedError("write your pallas kernel here")


# speedup vs baseline: 13.8420x; 13.8420x over previous
import jax
import jax.numpy as jnp
from jax.experimental import pallas as pl
from jax.experimental.pallas import tpu as pltpu

_K_REP = 8
_K_SMOOTH = 16
_W_CHAMFER, _W_REP, _W_COV, _W_SMOOTH = 1.0, 0.1, 0.2, 0.05
_REP_THRESHOLD = 0.005

_N = 4096
_NP = 2048
_BQ = 256
_NBP = _N // _BQ
_NBC = _NP // _BQ
_GRID_I = _NBP + _NBC

_BIG = 1e30


def _sqdist(q3, t3):
    qn = jnp.sum(q3 * q3, axis=1, keepdims=True)
    pn = jnp.sum(t3 * t3, axis=0, keepdims=True)
    cross = jnp.dot(q3, t3, preferred_element_type=jnp.float32)
    sq = qn + pn - 2.0 * cross
    return jnp.where(sq > 1e-12, sq, 0.0)


def _loss_kernel(pred_q_ref, pred_all_ref, pred_t_ref, gt_t_ref, part_q_ref,
                 out_ref, p4, cmin, acc_s):
    i = pl.program_id(1)

    @pl.when(i == 0)
    def _init():
        pr = pred_all_ref[0]
        p4[:, 0:3] = pr
        p4[:, 3:4] = jnp.sum(pr * pr, axis=1, keepdims=True)
        p4[:, 4:8] = jnp.zeros_like(p4[:, 4:8])
        cmin[...] = jnp.full_like(cmin[...], _BIG)
        for k in range(6):
            acc_s[k] = 0.0

    @pl.when(i < _NBP)
    def _pred_phase():
        q3 = pred_q_ref[0]

        sq_pg = _sqdist(q3, gt_t_ref[0])
        rowmin = jnp.min(sq_pg, axis=1, keepdims=True)
        acc_s[0] = acc_s[0] + jnp.sum(jnp.sqrt(rowmin))
        bcol = jnp.min(sq_pg, axis=0, keepdims=True)
        cmin[0:1, :] = jnp.minimum(cmin[0:1, :], bcol)

        d = _sqdist(q3, pred_t_ref[0])
        cid = jax.lax.broadcasted_iota(jnp.int32, (_BQ, _N), 1)
        fiota = cid.astype(jnp.float32)
        selfidx = (jax.lax.broadcasted_iota(jnp.int32, (_BQ, 1), 0)
                   + i * _BQ).astype(jnp.float32)

        def body(t, carry):
            d, rep, cnt = carry
            m = jnp.min(d, axis=1, keepdims=True)
            j = jnp.min(jnp.where(d == m, fiota, float(_N)),
                        axis=1, keepdims=True)
            d = jnp.where(fiota == j, _BIG, d)
            nonself = j != selfidx
            use = jnp.logical_and(nonself, cnt < float(_K_REP))
            rep = rep + jnp.where(
                use, jnp.maximum(_REP_THRESHOLD - jnp.sqrt(m), 0.0), 0.0)
            cnt = cnt + jnp.where(nonself, 1.0, 0.0)
            return d, rep, cnt

        zero = jnp.zeros((_BQ, 1), jnp.float32)
        d, repv, _ = jax.lax.fori_loop(
            0, _K_SMOOTH, body, (d, zero, zero))
        acc_s[2] = acc_s[2] + jnp.sum(repv)

        smask = (d == _BIG).astype(jnp.float32)
        sums = jnp.dot(smask, p4[...], preferred_element_type=jnp.float32,
                       precision=jax.lax.Precision.HIGHEST)
        sumc = sums[:, 0:3]
        sumsq = sums[:, 3:4]
        var = (sumsq - jnp.sum(sumc * sumc, axis=1, keepdims=True)
               / float(_K_SMOOTH)) / float(_K_SMOOTH * 3 - 1)
        acc_s[3] = acc_s[3] + jnp.sum(var)

    @pl.when(i == _NBP - 1)
    def _finish_cols():
        acc_s[1] = acc_s[1] + jnp.sum(jnp.sqrt(cmin[0:1, :]))

    @pl.when(i >= _NBP)
    def _cov_phase():
        c3 = part_q_ref[0]
        sq_cp = _sqdist(c3, pred_t_ref[0])
        rmin = jnp.min(sq_cp, axis=1, keepdims=True)
        md = jnp.sqrt(rmin)
        mask = (jnp.sum(jnp.abs(c3), axis=1, keepdims=True)
                > 1e-6).astype(jnp.float32)
        acc_s[4] = acc_s[4] + jnp.sum(md * mask)
        acc_s[5] = acc_s[5] + jnp.sum(mask)

    @pl.when(i == _GRID_I - 1)
    def _write():
        lane = jax.lax.broadcasted_iota(jnp.int32, (1, 128), 1)
        r = jnp.zeros((1, 128), jnp.float32)
        for k in range(6):
            r = jnp.where(lane == k, acc_s[k], r)
        out_ref[0] = r


def kernel(pred, gt, partial):
    B, N, _ = pred.shape
    pred_t = jnp.transpose(pred, (0, 2, 1))
    gt_t = jnp.transpose(gt, (0, 2, 1))

    partials = pl.pallas_call(
        _loss_kernel,
        grid=(B, _GRID_I),
        in_specs=[
            pl.BlockSpec((1, _BQ, 3),
                         lambda b, i: (b, jnp.minimum(i, _NBP - 1), 0)),
            pl.BlockSpec((1, _N, 3), lambda b, i: (b, 0, 0)),
            pl.BlockSpec((1, 3, _N), lambda b, i: (b, 0, 0)),
            pl.BlockSpec((1, 3, _N), lambda b, i: (b, 0, 0)),
            pl.BlockSpec((1, _BQ, 3),
                         lambda b, i: (b, jnp.clip(i - _NBP, 0, _NBC - 1), 0)),
        ],
        out_specs=pl.BlockSpec((1, 1, 128), lambda b, i: (b, 0, 0)),
        out_shape=jax.ShapeDtypeStruct((B, 1, 128), jnp.float32),
        scratch_shapes=[
            pltpu.VMEM((_N, 8), jnp.float32),
            pltpu.VMEM((8, _N), jnp.float32),
            pltpu.SMEM((8,), jnp.float32),
        ],
        compiler_params=pltpu.CompilerParams(
            dimension_semantics=("parallel", "arbitrary"),
            vmem_limit_bytes=100 * 1024 * 1024,
        ),
    )(pred, pred, pred_t, gt_t, partial)

    p = partials[:, 0, :]
    denom = float(B * N)
    chamfer = (jnp.sum(p[:, 0]) + jnp.sum(p[:, 1])) / denom * _W_CHAMFER
    repulsion = jnp.sum(p[:, 2]) / (denom * _K_REP) * _W_REP
    smoothness = jnp.sum(p[:, 3]) / denom * _W_SMOOTH
    cnt = p[:, 5]
    per_b = jnp.where(cnt > 0, p[:, 4] / jnp.maximum(cnt, 1.0), 0.0)
    coverage = jnp.mean(per_b) * _W_COV
    total = chamfer + repulsion + coverage + smoothness
    return jnp.stack([chamfer, repulsion, coverage, smoothness, total])
